# speedup vs baseline: 1.7154x; 1.7154x over previous
import functools

import jax
import jax.numpy as jnp
from jax import lax
from jax.experimental import pallas as pl
from jax.experimental.pallas import tpu as pltpu
from jax.experimental.pallas import tpu_sc as plsc

_NC = 2
_NS = 16
_NW = _NC * _NS

_B = 1024 * 200
_D = 768
_BPW = _B // _NW
_CHUNK = 64
_NCHUNK = _BPW // _CHUNK

_mesh = plsc.VectorSubcoreMesh(core_axis_name="c", subcore_axis_name="s")


@functools.partial(
    pl.kernel,
    out_type=jax.ShapeDtypeStruct((_B, _D), jnp.float32),
    mesh=_mesh,
    scratch_types=[
        pltpu.VMEM((_NCHUNK, _CHUNK), jnp.int32),
        pltpu.VMEM((_CHUNK, _D), jnp.float32),
        pltpu.SemaphoreType.DMA,
    ],
)
def _gather_rows(table_hbm, idx_hbm, out_hbm, idx_v, rows_v, sem):
    wid = lax.axis_index("s") * _NC + lax.axis_index("c")
    base = wid * _BPW
    pltpu.sync_copy(idx_hbm.at[wid], idx_v)

    @pl.loop(0, _NCHUNK)
    def _chunk(j):
        pltpu.async_copy(table_hbm.at[idx_v.at[j]], rows_v, sem).wait()
        pltpu.sync_copy(rows_v, out_hbm.at[pl.ds(base + j * _CHUNK, _CHUNK)])


def kernel(x, table):
    idx = x.reshape(_NW, _NCHUNK, _CHUNK)
    out = _gather_rows(table, idx)
    return out.reshape(x.shape[0], x.shape[1], _D)


# speedup vs baseline: 1.8925x; 1.1032x over previous
import functools

import jax
import jax.numpy as jnp
from jax import lax
from jax.experimental import pallas as pl
from jax.experimental.pallas import tpu as pltpu
from jax.experimental.pallas import tpu_sc as plsc

_NC = 2
_NS = 16
_NW = _NC * _NS

_B = 1024 * 200
_D = 768
_BPW = _B // _NW
_CHUNK = 64
_NCHUNK = _BPW // _CHUNK

_mesh = plsc.VectorSubcoreMesh(core_axis_name="c", subcore_axis_name="s")


@functools.partial(
    pl.kernel,
    out_type=jax.ShapeDtypeStruct((_B, _D), jnp.float32),
    mesh=_mesh,
    scratch_types=[
        pltpu.VMEM((_NCHUNK, _CHUNK), jnp.int32),
        pltpu.VMEM((2, _CHUNK, _D), jnp.float32),
        pltpu.SemaphoreType.DMA,
        pltpu.SemaphoreType.DMA,
        pltpu.SemaphoreType.DMA,
        pltpu.SemaphoreType.DMA,
    ],
)
def _gather_rows(table_hbm, idx_hbm, out_hbm, idx_v, rows_v, g0, g1, s0, s1):
    wid = lax.axis_index("s") * _NC + lax.axis_index("c")
    base = wid * _BPW
    gsem = (g0, g1)
    ssem = (s0, s1)
    pltpu.sync_copy(idx_hbm.at[wid], idx_v)

    def gather(c, b):
        return pltpu.make_async_copy(
            table_hbm.at[idx_v.at[c]], rows_v.at[b], gsem[b]
        )

    def store(c, b):
        return pltpu.make_async_copy(
            rows_v.at[b], out_hbm.at[pl.ds(base + c * _CHUNK, _CHUNK)], ssem[b]
        )

    gather(0, 0).start()

    @pl.loop(0, _NCHUNK, step=2)
    def _pair(j):
        for b in range(2):
            c = j + b
            nb = 1 - b
            gather(c, b).wait()

            @pl.when(c >= 1)
            def _():
                store(c - 1, nb).wait()

            @pl.when(c + 1 < _NCHUNK)
            def _():
                gather(c + 1, nb).start()

            store(c, b).start()

    store(_NCHUNK - 1, (_NCHUNK - 1) % 2).wait()


def kernel(x, table):
    idx = x.reshape(_NW, _NCHUNK, _CHUNK)
    out = _gather_rows(table, idx)
    return out.reshape(x.shape[0], x.shape[1], _D)
